# Initial kernel scaffold; baseline (speedup 1.0000x reference)
import jax
import jax.numpy as jnp
from jax.experimental import pallas as pl


def kernel(x, pos, edge_index, params):
    raise NotImplementedError("write your pallas kernel here")


# speedup vs baseline: 26.2045x; 26.2045x over previous
import functools

import jax
import jax.numpy as jnp
import numpy as np
from jax import lax
from jax.experimental import pallas as pl
from jax.experimental.pallas import tpu as pltpu
from jax.experimental.pallas import tpu_sc as plsc

N0 = 50000
E = 800000
N0P = 50176
EP = 819200
ER = EP // 128
NEG_SLOPE = 0.01
M_LIST = [16384, 4096, 1024, 1024]
MP_LIST = [16512, 4224, 1152, 1152]
POOL_SIZES = [[5.0, 5.0, 10.0], [2.0, 2.0, 1.0], [2.0, 2.0, 1.0], [1.0, 1.0, 1.0]]
CHANNELS = [16, 32, 64, 64, 64]

NC, NS = 2, 16
NW = NC * NS

_MESH = plsc.VectorSubcoreMesh(
    core_axis_name="c", subcore_axis_name="s", num_cores=NC, num_subcores=NS)
_SC_PARAMS = pltpu.CompilerParams(needs_layout_passes=False,
                                  use_tc_tiling_on_sc=False)


def _wid():
    return lax.axis_index("c") * NS + lax.axis_index("s")


_EPT = EP // NW
_K1CH = _EPT // 4


_SL0 = N0P // NS


@functools.partial(
    pl.kernel, mesh=_MESH, compiler_params=_SC_PARAMS,
    out_type=jax.ShapeDtypeStruct((NW * N0P,), jnp.float32),
    scratch_types=[
        pltpu.VMEM((N0P,), jnp.float32),
        pltpu.VMEM((N0P,), jnp.float32),
        pltpu.VMEM((_K1CH,), jnp.int32),
        pltpu.VMEM((_K1CH,), jnp.int32),
        pltpu.SemaphoreType.DMA,
    ],
)
def _sc_seg0(x_hbm, e_hbm, out_hbm, xv, accv, sv, dv, sem):
    wid = _wid()

    def zero(i, carry):
        accv[pl.ds(i * 16, 16)] = jnp.zeros((16,), jnp.float32)
        return carry
    lax.fori_loop(0, N0P // 16, zero, 0)

    pltpu.sync_copy(x_hbm, xv)
    base = wid * _EPT

    def chunk(c, carry):
        cb = base + c * _K1CH
        pltpu.sync_copy(e_hbm.at[pl.ds(cb, _K1CH)], sv)
        pltpu.sync_copy(e_hbm.at[pl.ds(EP + cb, _K1CH)], dv)

        def body(i, carry2):
            sidx = sv[pl.ds(i * 16, 16)]
            vals = plsc.load_gather(xv, [sidx])
            didx = dv[pl.ds(i * 16, 16)]
            plsc.addupdate_scatter(accv, [didx], vals)
            return carry2
        lax.fori_loop(0, _K1CH // 16, body, 0)
        return carry
    lax.fori_loop(0, _EPT // _K1CH, chunk, 0)

    pltpu.sync_copy(accv, out_hbm.at[pl.ds(wid * N0P, N0P)])


def _make_remap(tab_n):
    tot = 2 * EP
    per_tile = tot // NW
    ch = per_tile // 4

    @functools.partial(
        pl.kernel, mesh=_MESH, compiler_params=_SC_PARAMS,
        out_type=jax.ShapeDtypeStruct((tot,), jnp.int32),
        scratch_types=[
            pltpu.VMEM((tab_n,), jnp.int32),
            pltpu.VMEM((ch,), jnp.int32),
            pltpu.VMEM((ch,), jnp.int32),
            pltpu.SemaphoreType.DMA,
        ],
    )
    def remap(tab_hbm, e_hbm, out_hbm, tv, inb, outb, sem):
        wid = _wid()
        pltpu.sync_copy(tab_hbm, tv)
        base = wid * per_tile

        def chunk(c, carry):
            cb = base + c * ch
            pltpu.sync_copy(e_hbm.at[pl.ds(cb, ch)], inb)

            def body(i, carry2):
                idx = inb[pl.ds(i * 16, 16)]
                outb[pl.ds(i * 16, 16)] = plsc.load_gather(tv, [idx])
                return carry2
            lax.fori_loop(0, ch // 16, body, 0)
            pltpu.sync_copy(outb, out_hbm.at[pl.ds(cb, ch)])
            return carry
        lax.fori_loop(0, 4, chunk, 0)

    return remap


def _make_pool(np_rows, mp, d):
    nchunks = np_rows // 128
    outer = (nchunks + NW - 1) // NW

    @functools.partial(
        pl.kernel, mesh=_MESH, compiler_params=_SC_PARAMS,
        out_type=jax.ShapeDtypeStruct((NC, mp, d), jnp.float32),
        scratch_types=[
            pltpu.VMEM((1, 128), jnp.int32),
            pltpu.VMEM((128, d), jnp.float32),
            pltpu.VMEM_SHARED((mp, d), jnp.float32),
            pltpu.SemaphoreType.DMA,
        ],
    )
    def pool(rows_hbm, vid_hbm, z_hbm, out_hbm, vidb, rowsb, acc_sh, sem):
        cid = lax.axis_index("c")
        sid = lax.axis_index("s")
        wid = cid * NS + sid

        @pl.when(sid == 0)
        def _():
            pltpu.sync_copy(z_hbm, acc_sh)
        plsc.subcore_barrier()

        def chunk(jj, carry):
            j = jj * NW + wid

            @pl.when(j < nchunks)
            def _():
                pltpu.sync_copy(vid_hbm.at[pl.ds(j * 128, 128)], vidb.at[0])
                pltpu.sync_copy(rows_hbm.at[pl.ds(j * 128, 128)], rowsb)
                pltpu.sync_copy(rowsb, acc_sh.at[vidb.at[0]], add=True)
            return carry
        lax.fori_loop(0, outer, chunk, 0)

        plsc.subcore_barrier()

        @pl.when(sid == 0)
        def _():
            pltpu.sync_copy(acc_sh, out_hbm.at[cid])

    return pool


_CROWS = ER // NW


def _make_conv(mp, d):
    @functools.partial(
        pl.kernel, mesh=_MESH, compiler_params=_SC_PARAMS,
        out_type=jax.ShapeDtypeStruct((NC, mp, d), jnp.float32),
        scratch_types=[
            pltpu.VMEM((_CROWS, 128), jnp.int32),
            pltpu.VMEM((_CROWS, 128), jnp.int32),
            pltpu.VMEM((128, d), jnp.float32),
            pltpu.VMEM((128, d), jnp.float32),
            pltpu.VMEM_SHARED((mp, d), jnp.float32),
            pltpu.SemaphoreType.DMA((2,)),
            pltpu.SemaphoreType.DMA((2,)),
            pltpu.SemaphoreType.DMA,
        ],
    )
    def conv(xp_hbm, e_hbm, z_hbm, out_hbm, sidxb, didxb, buf0, buf1,
             acc_sh, semg, sems, sem):
        cid = lax.axis_index("c")
        sid = lax.axis_index("s")
        wid = cid * NS + sid
        rbase = wid * _CROWS

        @pl.when(sid == 0)
        def _():
            pltpu.sync_copy(z_hbm, acc_sh)

        pltpu.sync_copy(e_hbm.at[0, pl.ds(rbase, _CROWS)], sidxb, )
        pltpu.sync_copy(e_hbm.at[1, pl.ds(rbase, _CROWS)], didxb, )
        plsc.subcore_barrier()

        bufs = (buf0, buf1)

        def fire_gather(t, b):
            pltpu.async_copy(xp_hbm.at[sidxb.at[t]], b, semg.at[t % 2])

        def step(t, carry):
            @pl.when(t < _CROWS)
            def _():
                @pl.when(t >= 2)
                def _():
                    pltpu.make_async_copy(
                        bufs[0], acc_sh.at[didxb.at[t - 2]], sems.at[t % 2]
                    ).wait()

                @pl.when(t % 2 == 0)
                def _():
                    fire_gather(t, buf0)

                @pl.when(t % 2 == 1)
                def _():
                    fire_gather(t, buf1)

            @pl.when(t >= 1)
            def _():
                j = t - 1

                @pl.when(j % 2 == 0)
                def _():
                    pltpu.make_async_copy(
                        xp_hbm.at[sidxb.at[j]], buf0, semg.at[j % 2]).wait()
                    pltpu.async_copy(
                        buf0, acc_sh.at[didxb.at[j]], sems.at[j % 2], add=True)

                @pl.when(j % 2 == 1)
                def _():
                    pltpu.make_async_copy(
                        xp_hbm.at[sidxb.at[j]], buf1, semg.at[j % 2]).wait()
                    pltpu.async_copy(
                        buf1, acc_sh.at[didxb.at[j]], sems.at[j % 2], add=True)
            return carry
        lax.fori_loop(0, _CROWS + 1, step, 0)

        pltpu.make_async_copy(
            bufs[0], acc_sh.at[didxb.at[_CROWS - 2]], sems.at[_CROWS % 2]).wait()
        pltpu.make_async_copy(
            bufs[0], acc_sh.at[didxb.at[_CROWS - 1]], sems.at[(_CROWS - 1) % 2]).wait()

        plsc.subcore_barrier()

        @pl.when(sid == 0)
        def _():
            pltpu.sync_copy(acc_sh, out_hbm.at[cid])

    return conv


def _hash_vid_t(ppT, csize, m, nreal, ncols):
    s0, s1, s2 = (np.float32(csize[0]), np.float32(csize[1]),
                  np.float32(csize[2]))
    v0 = jnp.floor(ppT[0:1, :] / s0).astype(jnp.int32)
    v1 = jnp.floor(ppT[1:2, :] / s1).astype(jnp.int32)
    v2 = jnp.floor(ppT[2:3, :] / s2).astype(jnp.int32)
    h = v0 * 73856093 + v1 * 19349663 + v2 * 83492791
    vid = jnp.mod(h, m)
    col = lax.broadcasted_iota(jnp.int32, (1, ncols), 1)
    return jnp.where(col < nreal, vid, m)


def _bn_leaky_t(hT, nreal, gcol, becol):
    hr = hT[:, :nreal]
    mean = jnp.mean(hr, axis=1, keepdims=True)
    var = jnp.mean((hr - mean) ** 2, axis=1, keepdims=True)
    hn = (hT - mean) / jnp.sqrt(var + 1e-5) * gcol + becol
    return jnp.where(hn > 0, hn, NEG_SLOPE * hn)


def _tc_call(fn, out_shapes):
    return pl.pallas_call(fn, out_shape=out_shapes)


def _tc0(xT, posT, segp, wmT, wsT, bc, gc, bec, csize1):
    def body(x_r, pos_r, segp_r, wm_r, ws_r, b_r, g_r, be_r,
             fused_r, vid_r):
        segT = jnp.sum(segp_r[...], axis=0, keepdims=True)
        hT = wm_r[...] * segT + ws_r[...] * x_r[...] + b_r[...]
        hT = _bn_leaky_t(hT, N0, g_r[...], be_r[...])
        fused_r[...] = jnp.concatenate(
            [hT, pos_r[...], jnp.ones((1, N0P), jnp.float32),
             jnp.zeros((12, N0P), jnp.float32)], axis=0)
        vid_r[...] = _hash_vid_t(pos_r[...], csize1, M_LIST[0], N0, N0P)

    return _tc_call(body, (
        jax.ShapeDtypeStruct((32, N0P), jnp.float32),
        jax.ShapeDtypeStruct((1, N0P), jnp.int32),
    ))(xT, posT, segp, wmT, wsT, bc, gc, bec)


def _tc_a(poolpT, d, m, mp, csize_next, m_next, emit_vid):
    def body(poolp_r, *rest):
        if emit_vid:
            xp_r, pp_r, vid_r = rest
        else:
            xp_r, pp_r = rest
        acc = poolp_r[0] + poolp_r[1]
        cnt = jnp.maximum(acc[d + 3:d + 4, :], 1.0)
        xpT = acc[:d] / cnt
        ppT = acc[d:d + 3] / cnt
        xp_r[...] = xpT
        pp_r[...] = ppT
        if emit_vid:
            vid_r[...] = _hash_vid_t(ppT, csize_next, m_next, m, mp)

    outs = [jax.ShapeDtypeStruct((d, mp), jnp.float32),
            jax.ShapeDtypeStruct((3, mp), jnp.float32)]
    if emit_vid:
        outs.append(jax.ShapeDtypeStruct((1, mp), jnp.int32))
    return _tc_call(body, tuple(outs))(poolpT)


def _tc_b(aggpT, xpT, ppT, wmT, wsT, bc, gc, bec, m, mp, dout, last):
    def body(aggp_r, xp_r, pp_r, wm_r, ws_r, b_r, g_r, be_r, out_r):
        aggT = aggp_r[0] + aggp_r[1]
        hT = (jnp.dot(wm_r[...], aggT, preferred_element_type=jnp.float32)
              + jnp.dot(ws_r[...], xp_r[...],
                        preferred_element_type=jnp.float32)
              + b_r[...])
        hT = _bn_leaky_t(hT, m, g_r[...], be_r[...])
        if last:
            out_r[...] = hT
        else:
            out_r[...] = jnp.concatenate(
                [hT, pp_r[...], jnp.ones((1, mp), jnp.float32),
                 jnp.zeros((12, mp), jnp.float32)], axis=0)

    width = dout if last else dout + 16
    return _tc_call(body, jax.ShapeDtypeStruct((width, mp), jnp.float32))(
        aggpT, xpT, ppT, wmT, wsT, bc, gc, bec)


def kernel(x, pos, edge_index, params):
    xf = jnp.pad(x[:, 0], (0, N0P - N0))
    posp = jnp.pad(pos, ((0, N0P - N0), (0, 0)))
    e = jnp.pad(edge_index, ((0, 0), (0, EP - E)),
                constant_values=N0P - 1).astype(jnp.int32)
    eflat = e.reshape(2 * EP)

    csizes = []
    cs = np.ones(3, dtype=np.float32)
    for ps in POOL_SIZES:
        cs = cs * np.asarray(ps, dtype=np.float32)
        csizes.append(cs.copy())

    segp = _sc_seg0(xf, eflat).reshape(NW, N0P)
    wm, ws, b, g, be = params[0]
    fusedT, vid = _tc0(xf.reshape(1, N0P), posp.T, segp,
                       wm.T, ws.T, b[:, None], g[:, None], be[:, None],
                       csizes[0])

    nrows = N0P
    for i in range(4):
        m, mp, d = M_LIST[i], MP_LIST[i], CHANNELS[i]
        dout = CHANNELS[i + 1]
        vid_flat = vid.reshape(nrows)
        poolp = _make_pool(nrows, mp, d + 16)(
            fusedT.T, vid_flat, jnp.zeros((mp, d + 16), jnp.float32))
        eflat = _make_remap(nrows)(vid_flat, eflat)
        emit_vid = i < 3
        poolpT = poolp.transpose(0, 2, 1)
        if emit_vid:
            xpT, ppT, vid = _tc_a(poolpT, d, m, mp, csizes[i + 1],
                                  M_LIST[i + 1], True)
        else:
            xpT, ppT = _tc_a(poolpT, d, m, mp, None, None, False)
        e3 = eflat.reshape(2, ER, 128)
        aggp = _make_conv(mp, d)(xpT.T, e3, jnp.zeros((mp, d), jnp.float32))
        wm, ws, b, g, be = params[i + 1]
        fusedT = _tc_b(aggp.transpose(0, 2, 1), xpT, ppT, wm.T, ws.T,
                       b[:, None], g[:, None], be[:, None], m, mp, dout,
                       last=(i == 3))
        nrows = mp

    return fusedT.T[:1024]


# speedup vs baseline: 26.2813x; 1.0029x over previous
import functools

import jax
import jax.numpy as jnp
import numpy as np
from jax import lax
from jax.experimental import pallas as pl
from jax.experimental.pallas import tpu as pltpu
from jax.experimental.pallas import tpu_sc as plsc

N0 = 50000
E = 800000
N0P = 50176
EP = 819200
ER = EP // 128
NEG_SLOPE = 0.01
M_LIST = [16384, 4096, 1024, 1024]
MP_LIST = [16512, 4224, 1152, 1152]
POOL_SIZES = [[5.0, 5.0, 10.0], [2.0, 2.0, 1.0], [2.0, 2.0, 1.0], [1.0, 1.0, 1.0]]
CHANNELS = [16, 32, 64, 64, 64]

NC, NS = 2, 16
NW = NC * NS

_MESH = plsc.VectorSubcoreMesh(
    core_axis_name="c", subcore_axis_name="s", num_cores=NC, num_subcores=NS)
_SC_PARAMS = pltpu.CompilerParams(needs_layout_passes=False,
                                  use_tc_tiling_on_sc=False)


def _wid():
    return lax.axis_index("c") * NS + lax.axis_index("s")


_EPT = EP // NW
_K1CH = _EPT // 4


_SL0 = N0P // NS


@functools.partial(
    pl.kernel, mesh=_MESH, compiler_params=_SC_PARAMS,
    out_type=jax.ShapeDtypeStruct((NW * N0P,), jnp.float32),
    scratch_types=[
        pltpu.VMEM((N0P,), jnp.float32),
        pltpu.VMEM((N0P,), jnp.float32),
        pltpu.VMEM((_K1CH,), jnp.int32),
        pltpu.VMEM((_K1CH,), jnp.int32),
        pltpu.SemaphoreType.DMA,
    ],
)
def _sc_seg0(x_hbm, e_hbm, out_hbm, xv, accv, sv, dv, sem):
    wid = _wid()

    def zero(i, carry):
        accv[pl.ds(i * 16, 16)] = jnp.zeros((16,), jnp.float32)
        return carry
    lax.fori_loop(0, N0P // 16, zero, 0)

    pltpu.sync_copy(x_hbm, xv)
    base = wid * _EPT

    def chunk(c, carry):
        cb = base + c * _K1CH
        pltpu.sync_copy(e_hbm.at[pl.ds(cb, _K1CH)], sv)
        pltpu.sync_copy(e_hbm.at[pl.ds(EP + cb, _K1CH)], dv)

        def body(i, carry2):
            sidx = sv[pl.ds(i * 16, 16)]
            vals = plsc.load_gather(xv, [sidx])
            didx = dv[pl.ds(i * 16, 16)]
            plsc.addupdate_scatter(accv, [didx], vals)
            return carry2
        lax.fori_loop(0, _K1CH // 16, body, 0)
        return carry
    lax.fori_loop(0, _EPT // _K1CH, chunk, 0)

    pltpu.sync_copy(accv, out_hbm.at[pl.ds(wid * N0P, N0P)])


def _make_remap(tab_n):
    tot = 2 * EP
    per_tile = tot // NW
    ch = per_tile // 4

    @functools.partial(
        pl.kernel, mesh=_MESH, compiler_params=_SC_PARAMS,
        out_type=jax.ShapeDtypeStruct((tot,), jnp.int32),
        scratch_types=[
            pltpu.VMEM((tab_n,), jnp.int32),
            pltpu.VMEM((ch,), jnp.int32),
            pltpu.VMEM((ch,), jnp.int32),
            pltpu.SemaphoreType.DMA,
        ],
    )
    def remap(tab_hbm, e_hbm, out_hbm, tv, inb, outb, sem):
        wid = _wid()
        pltpu.sync_copy(tab_hbm, tv)
        base = wid * per_tile

        def chunk(c, carry):
            cb = base + c * ch
            pltpu.sync_copy(e_hbm.at[pl.ds(cb, ch)], inb)

            def body(i, carry2):
                idx = inb[pl.ds(i * 16, 16)]
                outb[pl.ds(i * 16, 16)] = plsc.load_gather(tv, [idx])
                return carry2
            lax.fori_loop(0, ch // 16, body, 0)
            pltpu.sync_copy(outb, out_hbm.at[pl.ds(cb, ch)])
            return carry
        lax.fori_loop(0, 4, chunk, 0)

    return remap


def _make_pool(np_rows, mp, d):
    nchunks = np_rows // 128
    outer = (nchunks + NW - 1) // NW

    @functools.partial(
        pl.kernel, mesh=_MESH, compiler_params=_SC_PARAMS,
        out_type=jax.ShapeDtypeStruct((NC, mp, d), jnp.float32),
        scratch_types=[
            pltpu.VMEM((1, 128), jnp.int32),
            pltpu.VMEM((128, d), jnp.float32),
            pltpu.VMEM_SHARED((mp, d), jnp.float32),
            pltpu.SemaphoreType.DMA,
        ],
    )
    def pool(rows_hbm, vid_hbm, z_hbm, out_hbm, vidb, rowsb, acc_sh, sem):
        cid = lax.axis_index("c")
        sid = lax.axis_index("s")
        wid = cid * NS + sid

        @pl.when(sid == 0)
        def _():
            pltpu.sync_copy(z_hbm, acc_sh)
        plsc.subcore_barrier()

        def chunk(jj, carry):
            j = jj * NW + wid

            @pl.when(j < nchunks)
            def _():
                pltpu.sync_copy(vid_hbm.at[pl.ds(j * 128, 128)], vidb.at[0])
                pltpu.sync_copy(rows_hbm.at[pl.ds(j * 128, 128)], rowsb)
                pltpu.sync_copy(rowsb, acc_sh.at[vidb.at[0]], add=True)
            return carry
        lax.fori_loop(0, outer, chunk, 0)

        plsc.subcore_barrier()

        @pl.when(sid == 0)
        def _():
            pltpu.sync_copy(acc_sh, out_hbm.at[cid])

    return pool


_CROWS = ER // NW
_NB = 8
_KA = 4


def _make_conv(mp, d):
    @functools.partial(
        pl.kernel, mesh=_MESH, compiler_params=_SC_PARAMS,
        out_type=jax.ShapeDtypeStruct((NC, mp, d), jnp.float32),
        scratch_types=[
            pltpu.VMEM((_CROWS, 128), jnp.int32),
            pltpu.VMEM((_CROWS, 128), jnp.int32),
            [pltpu.VMEM((128, d), jnp.float32) for _ in range(_NB)],
            pltpu.VMEM_SHARED((mp, d), jnp.float32),
            pltpu.SemaphoreType.DMA((_NB,)),
            pltpu.SemaphoreType.DMA((_NB,)),
        ],
    )
    def conv(xp_hbm, e_hbm, z_hbm, out_hbm, sidxb, didxb, bufs,
             acc_sh, semg, sems):
        cid = lax.axis_index("c")
        sid = lax.axis_index("s")
        wid = cid * NS + sid
        rbase = wid * _CROWS

        @pl.when(sid == 0)
        def _():
            pltpu.sync_copy(z_hbm, acc_sh)

        pltpu.sync_copy(e_hbm.at[0, pl.ds(rbase, _CROWS)], sidxb)
        pltpu.sync_copy(e_hbm.at[1, pl.ds(rbase, _CROWS)], didxb)
        plsc.subcore_barrier()

        def on_slot(slot, fn):
            for k in range(_NB):
                @pl.when(slot == k)
                def _(k=k):
                    fn(k)

        for k in range(_KA):
            pltpu.async_copy(xp_hbm.at[sidxb.at[k]], bufs[k], semg.at[k])

        def step(j, carry):
            b = j % _NB

            def consume(k):
                pltpu.make_async_copy(
                    xp_hbm.at[sidxb.at[j]], bufs[k], semg.at[b]).wait()
                pltpu.async_copy(
                    bufs[k], acc_sh.at[didxb.at[j]], sems.at[b], add=True)
            on_slot(b, consume)

            g = j + _KA

            @pl.when(g < _CROWS)
            def _():
                bg = g % _NB

                def refill(k):
                    @pl.when(g >= _NB)
                    def _():
                        pltpu.make_async_copy(
                            bufs[k], acc_sh.at[didxb.at[j]],
                            sems.at[bg]).wait()
                    pltpu.async_copy(
                        xp_hbm.at[sidxb.at[g]], bufs[k], semg.at[bg])
                on_slot(bg, refill)
            return carry
        lax.fori_loop(0, _CROWS, step, 0)

        for k in range(_NB):
            pltpu.make_async_copy(
                bufs[k], acc_sh.at[didxb.at[k]], sems.at[k]).wait()

        plsc.subcore_barrier()

        @pl.when(sid == 0)
        def _():
            pltpu.sync_copy(acc_sh, out_hbm.at[cid])

    return conv


def _hash_vid_t(ppT, csize, m, nreal, ncols):
    s0, s1, s2 = (np.float32(csize[0]), np.float32(csize[1]),
                  np.float32(csize[2]))
    v0 = jnp.floor(ppT[0:1, :] / s0).astype(jnp.int32)
    v1 = jnp.floor(ppT[1:2, :] / s1).astype(jnp.int32)
    v2 = jnp.floor(ppT[2:3, :] / s2).astype(jnp.int32)
    h = v0 * 73856093 + v1 * 19349663 + v2 * 83492791
    vid = jnp.mod(h, m)
    col = lax.broadcasted_iota(jnp.int32, (1, ncols), 1)
    return jnp.where(col < nreal, vid, m)


def _bn_leaky_t(hT, nreal, gcol, becol):
    hr = hT[:, :nreal]
    mean = jnp.mean(hr, axis=1, keepdims=True)
    var = jnp.mean((hr - mean) ** 2, axis=1, keepdims=True)
    hn = (hT - mean) / jnp.sqrt(var + 1e-5) * gcol + becol
    return jnp.where(hn > 0, hn, NEG_SLOPE * hn)


def _tc_call(fn, out_shapes):
    return pl.pallas_call(fn, out_shape=out_shapes)


def _tc0(xT, posT, segp, wmT, wsT, bc, gc, bec, csize1):
    def body(x_r, pos_r, segp_r, wm_r, ws_r, b_r, g_r, be_r,
             fused_r, vid_r):
        segT = jnp.sum(segp_r[...], axis=0, keepdims=True)
        hT = wm_r[...] * segT + ws_r[...] * x_r[...] + b_r[...]
        hT = _bn_leaky_t(hT, N0, g_r[...], be_r[...])
        fused_r[...] = jnp.concatenate(
            [hT, pos_r[...], jnp.ones((1, N0P), jnp.float32),
             jnp.zeros((12, N0P), jnp.float32)], axis=0)
        vid_r[...] = _hash_vid_t(pos_r[...], csize1, M_LIST[0], N0, N0P)

    return _tc_call(body, (
        jax.ShapeDtypeStruct((32, N0P), jnp.float32),
        jax.ShapeDtypeStruct((1, N0P), jnp.int32),
    ))(xT, posT, segp, wmT, wsT, bc, gc, bec)


def _tc_a(poolpT, d, m, mp, csize_next, m_next, emit_vid):
    def body(poolp_r, *rest):
        if emit_vid:
            xp_r, pp_r, vid_r = rest
        else:
            xp_r, pp_r = rest
        acc = poolp_r[0] + poolp_r[1]
        cnt = jnp.maximum(acc[d + 3:d + 4, :], 1.0)
        xpT = acc[:d] / cnt
        ppT = acc[d:d + 3] / cnt
        xp_r[...] = xpT
        pp_r[...] = ppT
        if emit_vid:
            vid_r[...] = _hash_vid_t(ppT, csize_next, m_next, m, mp)

    outs = [jax.ShapeDtypeStruct((d, mp), jnp.float32),
            jax.ShapeDtypeStruct((3, mp), jnp.float32)]
    if emit_vid:
        outs.append(jax.ShapeDtypeStruct((1, mp), jnp.int32))
    return _tc_call(body, tuple(outs))(poolpT)


def _tc_b(aggpT, xpT, ppT, wmT, wsT, bc, gc, bec, m, mp, dout, last):
    def body(aggp_r, xp_r, pp_r, wm_r, ws_r, b_r, g_r, be_r, out_r):
        aggT = aggp_r[0] + aggp_r[1]
        hT = (jnp.dot(wm_r[...], aggT, preferred_element_type=jnp.float32)
              + jnp.dot(ws_r[...], xp_r[...],
                        preferred_element_type=jnp.float32)
              + b_r[...])
        hT = _bn_leaky_t(hT, m, g_r[...], be_r[...])
        if last:
            out_r[...] = hT
        else:
            out_r[...] = jnp.concatenate(
                [hT, pp_r[...], jnp.ones((1, mp), jnp.float32),
                 jnp.zeros((12, mp), jnp.float32)], axis=0)

    width = dout if last else dout + 16
    return _tc_call(body, jax.ShapeDtypeStruct((width, mp), jnp.float32))(
        aggpT, xpT, ppT, wmT, wsT, bc, gc, bec)


def kernel(x, pos, edge_index, params):
    xf = jnp.pad(x[:, 0], (0, N0P - N0))
    posp = jnp.pad(pos, ((0, N0P - N0), (0, 0)))
    e = jnp.pad(edge_index, ((0, 0), (0, EP - E)),
                constant_values=N0P - 1).astype(jnp.int32)
    eflat = e.reshape(2 * EP)

    csizes = []
    cs = np.ones(3, dtype=np.float32)
    for ps in POOL_SIZES:
        cs = cs * np.asarray(ps, dtype=np.float32)
        csizes.append(cs.copy())

    segp = _sc_seg0(xf, eflat).reshape(NW, N0P)
    wm, ws, b, g, be = params[0]
    fusedT, vid = _tc0(xf.reshape(1, N0P), posp.T, segp,
                       wm.T, ws.T, b[:, None], g[:, None], be[:, None],
                       csizes[0])

    nrows = N0P
    for i in range(4):
        m, mp, d = M_LIST[i], MP_LIST[i], CHANNELS[i]
        dout = CHANNELS[i + 1]
        vid_flat = vid.reshape(nrows)
        poolp = _make_pool(nrows, mp, d + 16)(
            fusedT.T, vid_flat, jnp.zeros((mp, d + 16), jnp.float32))
        eflat = _make_remap(nrows)(vid_flat, eflat)
        emit_vid = i < 3
        poolpT = poolp.transpose(0, 2, 1)
        if emit_vid:
            xpT, ppT, vid = _tc_a(poolpT, d, m, mp, csizes[i + 1],
                                  M_LIST[i + 1], True)
        else:
            xpT, ppT = _tc_a(poolpT, d, m, mp, None, None, False)
        e3 = eflat.reshape(2, ER, 128)
        aggp = _make_conv(mp, d)(xpT.T, e3, jnp.zeros((mp, d), jnp.float32))
        wm, ws, b, g, be = params[i + 1]
        fusedT = _tc_b(aggp.transpose(0, 2, 1), xpT, ppT, wm.T, ws.T,
                       b[:, None], g[:, None], be[:, None], m, mp, dout,
                       last=(i == 3))
        nrows = mp

    return fusedT.T[:1024]
